# baseline (speedup 1.0000x reference)
import jax
import jax.numpy as jnp
from jax import lax
from jax.experimental import pallas as pl
from jax.experimental.pallas import tpu as pltpu

_N = 16384
_C = 1000
_CB = 2048
_NB = _N // _CB


def _body(y_ref, x_ref, o_ref):
    y = y_ref[0, 0, :]
    x = x_ref[...]
    rows = lax.broadcasted_iota(jnp.int32, (_C, _CB), 0)
    sel = jnp.where(rows == y[None, :], x, 0.0)
    o_ref[0, 0, :] = jnp.sum(sel, axis=0)


def kernel(logits, y):
    lt = logits.T
    y2 = y.astype(jnp.int32).reshape(_NB, 1, _CB)
    out = pl.pallas_call(
        _body,
        grid=(_NB,),
        in_specs=[
            pl.BlockSpec((1, 1, _CB), lambda i: (i, 0, 0)),
            pl.BlockSpec((_C, _CB), lambda i: (0, i)),
        ],
        out_specs=pl.BlockSpec((1, 1, _CB), lambda i: (i, 0, 0)),
        out_shape=jax.ShapeDtypeStruct((_NB, 1, _CB), jnp.float32),
        compiler_params=pltpu.CompilerParams(dimension_semantics=("parallel",)),
    )(y2, lt)
    return out.reshape(-1)


# speedup vs baseline: 1.0192x; 1.0192x over previous
import jax
import jax.numpy as jnp
from jax import lax
from jax.experimental import pallas as pl
from jax.experimental.pallas import tpu as pltpu

_N = 16384
_C = 1000
_CB = 2048
_NB = _N // _CB


def _body(y_ref, x_ref, o_ref):
    y = y_ref[0, 0, :]
    yb = y[None, :]
    riota = lax.broadcasted_iota(jnp.int32, (8, _CB), 0)
    acc = jnp.zeros((8, _CB), jnp.float32)
    for g in range(_C // 8):
        x8 = x_ref[pl.ds(g * 8, 8), :]
        acc = acc + jnp.where(riota + (g * 8) == yb, x8, 0.0)
    o_ref[0, 0, :] = jnp.sum(acc, axis=0)


def kernel(logits, y):
    lt = logits.T
    y2 = y.astype(jnp.int32).reshape(_NB, 1, _CB)
    out = pl.pallas_call(
        _body,
        grid=(_NB,),
        in_specs=[
            pl.BlockSpec((1, 1, _CB), lambda i: (i, 0, 0)),
            pl.BlockSpec((_C, _CB), lambda i: (0, i)),
        ],
        out_specs=pl.BlockSpec((1, 1, _CB), lambda i: (i, 0, 0)),
        out_shape=jax.ShapeDtypeStruct((_NB, 1, _CB), jnp.float32),
        compiler_params=pltpu.CompilerParams(dimension_semantics=("parallel",)),
    )(y2, lt)
    return out.reshape(-1)
